# initial kernel scaffold (unmeasured)
import jax
import jax.numpy as jnp
from jax import lax
from jax.experimental import pallas as pl
from jax.experimental.pallas import tpu as pltpu

N_DEV = 8
K_TILE = 512


def kernel(x, w_mat):
    m_per, k_dim = x.shape
    _, n_total = w_mat.shape
    n_per = n_total // N_DEV
    n_k = k_dim // K_TILE

    def body(x_ref, w_ref, out_ref, acc_ref, send_sems, recv_sems):
        j = pl.program_id(0)
        k = pl.program_id(1)
        my = lax.axis_index("i")

        @pl.when((j == 0) & (k == 0))
        def _entry_barrier():
            barrier = pltpu.get_barrier_semaphore()
            for p in range(N_DEV):
                pl.semaphore_signal(
                    barrier, inc=1,
                    device_id=(p,), device_id_type=pl.DeviceIdType.MESH,
                )
            pl.semaphore_wait(barrier, N_DEV)

        prod = jnp.dot(
            x_ref[:, pl.ds(k * K_TILE, K_TILE)],
            w_ref[:, :],
            preferred_element_type=jnp.float32,
        )

        @pl.when(k == 0)
        def _init():
            acc_ref[j] = prod

        @pl.when(k != 0)
        def _accum():
            acc_ref[j] += prod

        @pl.when(k == n_k - 1)
        def _emit():
            @pl.when(j == my)
            def _local():
                out_ref[pl.ds(my * m_per, m_per), :] = acc_ref[j]

            @pl.when(j != my)
            def _send():
                rdma = pltpu.make_async_remote_copy(
                    src_ref=acc_ref.at[j],
                    dst_ref=out_ref.at[pl.ds(my * m_per, m_per)],
                    send_sem=send_sems.at[j],
                    recv_sem=recv_sems.at[my],
                    device_id=(j,),
                    device_id_type=pl.DeviceIdType.MESH,
                )
                rdma.start()

        @pl.when((j == N_DEV - 1) & (k == n_k - 1))
        def _drain():
            for p in range(N_DEV):
                @pl.when(p != my)
                def _wait(p=p):
                    rdma = pltpu.make_async_remote_copy(
                        src_ref=acc_ref.at[p],
                        dst_ref=out_ref.at[pl.ds(p * m_per, m_per)],
                        send_sem=send_sems.at[p],
                        recv_sem=recv_sems.at[p],
                        device_id=(p,),
                        device_id_type=pl.DeviceIdType.MESH,
                    )
                    rdma.wait_recv()
                    rdma.wait_send()

    return pl.pallas_call(
        body,
        grid=(N_DEV, n_k),
        out_shape=jax.ShapeDtypeStruct((N_DEV * m_per, n_per), jnp.float32),
        in_specs=[
            pl.BlockSpec(memory_space=pltpu.VMEM),
            pl.BlockSpec((K_TILE, n_per), lambda j, k: (k, j)),
        ],
        out_specs=pl.BlockSpec(memory_space=pltpu.VMEM),
        scratch_shapes=[
            pltpu.VMEM((N_DEV, m_per, n_per), jnp.float32),
            pltpu.SemaphoreType.DMA((N_DEV,)),
            pltpu.SemaphoreType.DMA((N_DEV,)),
        ],
        compiler_params=pltpu.CompilerParams(
            dimension_semantics=("arbitrary", "arbitrary"),
            collective_id=0,
            vmem_limit_bytes=128 * 1024 * 1024,
        ),
    )(x, w_mat)


# baseline (device time: 306232 ns/iter reference)
import jax
import jax.numpy as jnp
from jax import lax
from jax.experimental import pallas as pl
from jax.experimental.pallas import tpu as pltpu

N_DEV = 8
K_TILE = 512


def kernel(x, w_mat):
    m_per, k_dim = x.shape
    _, n_total = w_mat.shape
    n_per = n_total // N_DEV
    n_k = k_dim // K_TILE

    def body(x_ref, w_ref, out_ref, acc_ref, send_sems, recv_sems):
        j = pl.program_id(0)
        k = pl.program_id(1)
        my = lax.axis_index("i")

        @pl.when((j == 0) & (k == 0))
        def _entry_barrier():
            barrier = pltpu.get_barrier_semaphore()
            for p in range(N_DEV):
                pl.semaphore_signal(
                    barrier, inc=1,
                    device_id=(p,), device_id_type=pl.DeviceIdType.MESH,
                )
            pl.semaphore_wait(barrier, N_DEV)

        prod = jnp.dot(
            x_ref[:, :],
            w_ref[:, :],
            preferred_element_type=jnp.float32,
        )

        @pl.when(k == 0)
        def _init():
            acc_ref[j] = prod

        @pl.when(k != 0)
        def _accum():
            acc_ref[j] += prod

        @pl.when(k == n_k - 1)
        def _emit():
            @pl.when(j == my)
            def _local():
                out_ref[pl.ds(my * m_per, m_per), :] = acc_ref[j]

            @pl.when(j != my)
            def _send():
                rdma = pltpu.make_async_remote_copy(
                    src_ref=acc_ref.at[j],
                    dst_ref=out_ref.at[pl.ds(my * m_per, m_per)],
                    send_sem=send_sems.at[j],
                    recv_sem=recv_sems.at[my],
                    device_id=(j,),
                    device_id_type=pl.DeviceIdType.MESH,
                )
                rdma.start()

        @pl.when((j == N_DEV - 1) & (k == n_k - 1))
        def _drain():
            for p in range(N_DEV):
                @pl.when(p != my)
                def _wait(p=p):
                    rdma = pltpu.make_async_remote_copy(
                        src_ref=acc_ref.at[p],
                        dst_ref=out_ref.at[pl.ds(p * m_per, m_per)],
                        send_sem=send_sems.at[p],
                        recv_sem=recv_sems.at[p],
                        device_id=(p,),
                        device_id_type=pl.DeviceIdType.MESH,
                    )
                    rdma.wait_recv()
                    rdma.wait_send()

    return pl.pallas_call(
        body,
        grid=(N_DEV, n_k),
        out_shape=jax.ShapeDtypeStruct((N_DEV * m_per, n_per), jnp.float32),
        in_specs=[
            pl.BlockSpec((m_per, K_TILE), lambda j, k: (0, k)),
            pl.BlockSpec((K_TILE, n_per), lambda j, k: (k, j)),
        ],
        out_specs=pl.BlockSpec(memory_space=pltpu.VMEM),
        scratch_shapes=[
            pltpu.VMEM((N_DEV, m_per, n_per), jnp.float32),
            pltpu.SemaphoreType.DMA((N_DEV,)),
            pltpu.SemaphoreType.DMA((N_DEV,)),
        ],
        compiler_params=pltpu.CompilerParams(
            dimension_semantics=("arbitrary", "arbitrary"),
            collective_id=0,
            vmem_limit_bytes=128 * 1024 * 1024,
        ),
    )(x, w_mat)


# device time: 217701 ns/iter; 1.4067x vs baseline; 1.4067x over previous
import jax
import jax.numpy as jnp
from jax import lax
from jax.experimental import pallas as pl
from jax.experimental.pallas import tpu as pltpu

N_DEV = 8
K_TILE = 512
N_SLOTS = 4


def kernel(x, w_mat):
    m_per, k_dim = x.shape
    _, n_total = w_mat.shape
    n_per = n_total // N_DEV
    n_k = k_dim // K_TILE

    my_out = lax.axis_index("i")
    perm = (my_out + jnp.arange(N_DEV, dtype=jnp.int32)) % N_DEV

    def body(perm_ref, x_ref, w_ref, out_ref, acc_ref, send_sems, recv_sems):
        j = pl.program_id(0)
        k = pl.program_id(1)
        my = lax.axis_index("i")
        slot = lax.rem(j, N_SLOTS)

        @pl.when((j == 0) & (k == 0))
        def _entry_barrier():
            barrier = pltpu.get_barrier_semaphore()
            for p in range(N_DEV):
                pl.semaphore_signal(
                    barrier, inc=1,
                    device_id=(p,), device_id_type=pl.DeviceIdType.MESH,
                )
            pl.semaphore_wait(barrier, N_DEV)

        @pl.when((j >= N_SLOTS + 1) & (k == 0))
        def _reuse_wait():
            rdma = pltpu.make_async_remote_copy(
                src_ref=acc_ref.at[slot],
                dst_ref=out_ref.at[pl.ds(0, m_per)],
                send_sem=send_sems.at[slot],
                recv_sem=recv_sems.at[0],
                device_id=(0,),
                device_id_type=pl.DeviceIdType.MESH,
            )
            rdma.wait_send()

        prod = jnp.dot(
            x_ref[:, pl.ds(k * K_TILE, K_TILE)],
            w_ref[:, :],
            preferred_element_type=jnp.float32,
        )

        @pl.when(k == 0)
        def _init():
            acc_ref[slot] = prod

        @pl.when(k != 0)
        def _accum():
            acc_ref[slot] += prod

        @pl.when(k == n_k - 1)
        def _emit():
            @pl.when(j == 0)
            def _local():
                out_ref[pl.ds(my * m_per, m_per), :] = acc_ref[slot]

            @pl.when(j != 0)
            def _send():
                target = perm_ref[j]
                rdma = pltpu.make_async_remote_copy(
                    src_ref=acc_ref.at[slot],
                    dst_ref=out_ref.at[pl.ds(my * m_per, m_per)],
                    send_sem=send_sems.at[slot],
                    recv_sem=recv_sems.at[my],
                    device_id=(target,),
                    device_id_type=pl.DeviceIdType.MESH,
                )
                rdma.start()

        @pl.when((j == N_DEV - 1) & (k == n_k - 1))
        def _drain():
            for p in range(N_DEV):
                @pl.when(p != my)
                def _wait_recv(p=p):
                    rdma = pltpu.make_async_remote_copy(
                        src_ref=acc_ref.at[0],
                        dst_ref=out_ref.at[pl.ds(p * m_per, m_per)],
                        send_sem=send_sems.at[0],
                        recv_sem=recv_sems.at[p],
                        device_id=(p,),
                        device_id_type=pl.DeviceIdType.MESH,
                    )
                    rdma.wait_recv()
            for s in range(N_SLOTS):
                rdma = pltpu.make_async_remote_copy(
                    src_ref=acc_ref.at[s],
                    dst_ref=out_ref.at[pl.ds(0, m_per)],
                    send_sem=send_sems.at[s],
                    recv_sem=recv_sems.at[0],
                    device_id=(0,),
                    device_id_type=pl.DeviceIdType.MESH,
                )
                rdma.wait_send()

    return pl.pallas_call(
        body,
        grid_spec=pltpu.PrefetchScalarGridSpec(
            num_scalar_prefetch=1,
            grid=(N_DEV, n_k),
            in_specs=[
                pl.BlockSpec(memory_space=pltpu.VMEM),
                pl.BlockSpec((K_TILE, n_per), lambda j, k, perm: (k, perm[j])),
            ],
            out_specs=pl.BlockSpec(memory_space=pltpu.VMEM),
            scratch_shapes=[
                pltpu.VMEM((N_SLOTS, m_per, n_per), jnp.float32),
                pltpu.SemaphoreType.DMA((N_SLOTS,)),
                pltpu.SemaphoreType.DMA((N_DEV,)),
            ],
        ),
        out_shape=jax.ShapeDtypeStruct((N_DEV * m_per, n_per), jnp.float32),
        compiler_params=pltpu.CompilerParams(
            dimension_semantics=("arbitrary", "arbitrary"),
            collective_id=0,
            vmem_limit_bytes=128 * 1024 * 1024,
        ),
    )(perm, x, w_mat)


# device time: 188603 ns/iter; 1.6237x vs baseline; 1.1543x over previous
import jax
import jax.numpy as jnp
from jax import lax
from jax.experimental import pallas as pl
from jax.experimental.pallas import tpu as pltpu

N_DEV = 8
K_TILE = 1024
N_SLOTS = 4


def kernel(x, w_mat):
    m_per, k_dim = x.shape
    _, n_total = w_mat.shape
    n_per = n_total // N_DEV
    n_k = k_dim // K_TILE

    my_out = lax.axis_index("i")
    perm = (my_out + jnp.arange(N_DEV, dtype=jnp.int32)) % N_DEV

    def body(perm_ref, x_ref, w_ref, out_ref, acc_ref, send_sems, recv_sems):
        j = pl.program_id(0)
        k = pl.program_id(1)
        my = lax.axis_index("i")
        slot = lax.rem(j, N_SLOTS)

        @pl.when((j == 0) & (k == 0))
        def _entry_barrier():
            barrier = pltpu.get_barrier_semaphore()
            for p in range(N_DEV):
                pl.semaphore_signal(
                    barrier, inc=1,
                    device_id=(p,), device_id_type=pl.DeviceIdType.MESH,
                )
            pl.semaphore_wait(barrier, N_DEV)

        @pl.when((j >= N_SLOTS + 1) & (k == 0))
        def _reuse_wait():
            rdma = pltpu.make_async_remote_copy(
                src_ref=acc_ref.at[slot],
                dst_ref=out_ref.at[pl.ds(0, m_per)],
                send_sem=send_sems.at[slot],
                recv_sem=recv_sems.at[0],
                device_id=(0,),
                device_id_type=pl.DeviceIdType.MESH,
            )
            rdma.wait_send()

        prod = jnp.dot(
            x_ref[:, pl.ds(k * K_TILE, K_TILE)],
            w_ref[:, :],
            preferred_element_type=jnp.float32,
        )

        @pl.when(k == 0)
        def _init():
            acc_ref[slot] = prod

        @pl.when(k != 0)
        def _accum():
            acc_ref[slot] += prod

        @pl.when(k == n_k - 1)
        def _emit():
            @pl.when(j == 0)
            def _local():
                out_ref[pl.ds(my * m_per, m_per), :] = acc_ref[slot]

            @pl.when(j != 0)
            def _send():
                target = perm_ref[j]
                rdma = pltpu.make_async_remote_copy(
                    src_ref=acc_ref.at[slot],
                    dst_ref=out_ref.at[pl.ds(my * m_per, m_per)],
                    send_sem=send_sems.at[slot],
                    recv_sem=recv_sems.at[my],
                    device_id=(target,),
                    device_id_type=pl.DeviceIdType.MESH,
                )
                rdma.start()

        @pl.when((j == N_DEV - 1) & (k == n_k - 1))
        def _drain():
            for p in range(N_DEV):
                @pl.when(p != my)
                def _wait_recv(p=p):
                    rdma = pltpu.make_async_remote_copy(
                        src_ref=acc_ref.at[0],
                        dst_ref=out_ref.at[pl.ds(p * m_per, m_per)],
                        send_sem=send_sems.at[0],
                        recv_sem=recv_sems.at[p],
                        device_id=(p,),
                        device_id_type=pl.DeviceIdType.MESH,
                    )
                    rdma.wait_recv()
            for s in range(N_SLOTS):
                rdma = pltpu.make_async_remote_copy(
                    src_ref=acc_ref.at[s],
                    dst_ref=out_ref.at[pl.ds(0, m_per)],
                    send_sem=send_sems.at[s],
                    recv_sem=recv_sems.at[0],
                    device_id=(0,),
                    device_id_type=pl.DeviceIdType.MESH,
                )
                rdma.wait_send()

    return pl.pallas_call(
        body,
        grid_spec=pltpu.PrefetchScalarGridSpec(
            num_scalar_prefetch=1,
            grid=(N_DEV, n_k),
            in_specs=[
                pl.BlockSpec(memory_space=pltpu.VMEM),
                pl.BlockSpec((K_TILE, n_per), lambda j, k, perm: (k, perm[j])),
            ],
            out_specs=pl.BlockSpec(memory_space=pltpu.VMEM),
            scratch_shapes=[
                pltpu.VMEM((N_SLOTS, m_per, n_per), jnp.float32),
                pltpu.SemaphoreType.DMA((N_SLOTS,)),
                pltpu.SemaphoreType.DMA((N_DEV,)),
            ],
        ),
        out_shape=jax.ShapeDtypeStruct((N_DEV * m_per, n_per), jnp.float32),
        compiler_params=pltpu.CompilerParams(
            dimension_semantics=("arbitrary", "arbitrary"),
            collective_id=0,
            vmem_limit_bytes=128 * 1024 * 1024,
        ),
    )(perm, x, w_mat)


# device time: 174699 ns/iter; 1.7529x vs baseline; 1.0796x over previous
import jax
import jax.numpy as jnp
from jax import lax
from jax.experimental import pallas as pl
from jax.experimental.pallas import tpu as pltpu

N_DEV = 8
K_TILE = 2048
N_SLOTS = 4


def kernel(x, w_mat):
    m_per, k_dim = x.shape
    _, n_total = w_mat.shape
    n_per = n_total // N_DEV
    n_k = k_dim // K_TILE

    my_out = lax.axis_index("i")
    perm = (my_out + jnp.arange(N_DEV, dtype=jnp.int32)) % N_DEV

    def body(perm_ref, x_ref, w_ref, out_ref, acc_ref, send_sems, recv_sems,
             local_sem):
        j = pl.program_id(0)
        k = pl.program_id(1)
        my = lax.axis_index("i")
        slot = lax.rem(j, N_SLOTS)

        @pl.when((j == 0) & (k == 0))
        def _entry_barrier():
            barrier = pltpu.get_barrier_semaphore()
            for p in range(N_DEV):
                pl.semaphore_signal(
                    barrier, inc=1,
                    device_id=(p,), device_id_type=pl.DeviceIdType.MESH,
                )
            pl.semaphore_wait(barrier, N_DEV)

        @pl.when((j >= N_SLOTS + 1) & (k == 0))
        def _reuse_wait():
            rdma = pltpu.make_async_remote_copy(
                src_ref=acc_ref.at[slot],
                dst_ref=out_ref.at[pl.ds(0, m_per)],
                send_sem=send_sems.at[slot],
                recv_sem=recv_sems.at[0],
                device_id=(0,),
                device_id_type=pl.DeviceIdType.MESH,
            )
            rdma.wait_send()

        prod = jnp.dot(
            x_ref[:, pl.ds(k * K_TILE, K_TILE)],
            w_ref[:, :],
            preferred_element_type=jnp.float32,
        )

        @pl.when(k == 0)
        def _init():
            acc_ref[slot] = prod

        @pl.when(k != 0)
        def _accum():
            acc_ref[slot] += prod

        @pl.when(k == n_k - 1)
        def _emit():
            @pl.when(j == 0)
            def _local():
                copy = pltpu.make_async_copy(
                    acc_ref.at[slot],
                    out_ref.at[pl.ds(my * m_per, m_per)],
                    local_sem,
                )
                copy.start()

            @pl.when(j != 0)
            def _send():
                target = perm_ref[j]
                rdma = pltpu.make_async_remote_copy(
                    src_ref=acc_ref.at[slot],
                    dst_ref=out_ref.at[pl.ds(my * m_per, m_per)],
                    send_sem=send_sems.at[slot],
                    recv_sem=recv_sems.at[my],
                    device_id=(target,),
                    device_id_type=pl.DeviceIdType.MESH,
                )
                rdma.start()

        @pl.when((j == N_DEV - 1) & (k == n_k - 1))
        def _drain():
            copy = pltpu.make_async_copy(
                acc_ref.at[0],
                out_ref.at[pl.ds(my * m_per, m_per)],
                local_sem,
            )
            copy.wait()
            for p in range(N_DEV):
                @pl.when(p != my)
                def _wait_recv(p=p):
                    rdma = pltpu.make_async_remote_copy(
                        src_ref=acc_ref.at[0],
                        dst_ref=out_ref.at[pl.ds(p * m_per, m_per)],
                        send_sem=send_sems.at[0],
                        recv_sem=recv_sems.at[p],
                        device_id=(p,),
                        device_id_type=pl.DeviceIdType.MESH,
                    )
                    rdma.wait_recv()
            for s in range(N_SLOTS):
                rdma = pltpu.make_async_remote_copy(
                    src_ref=acc_ref.at[s],
                    dst_ref=out_ref.at[pl.ds(0, m_per)],
                    send_sem=send_sems.at[s],
                    recv_sem=recv_sems.at[0],
                    device_id=(0,),
                    device_id_type=pl.DeviceIdType.MESH,
                )
                rdma.wait_send()

    return pl.pallas_call(
        body,
        grid_spec=pltpu.PrefetchScalarGridSpec(
            num_scalar_prefetch=1,
            grid=(N_DEV, n_k),
            in_specs=[
                pl.BlockSpec(memory_space=pltpu.VMEM),
                pl.BlockSpec((K_TILE, n_per), lambda j, k, perm: (k, perm[j])),
            ],
            out_specs=pl.BlockSpec(memory_space=pl.ANY),
            scratch_shapes=[
                pltpu.VMEM((N_SLOTS, m_per, n_per), jnp.float32),
                pltpu.SemaphoreType.DMA((N_SLOTS,)),
                pltpu.SemaphoreType.DMA((N_DEV,)),
                pltpu.SemaphoreType.DMA,
            ],
        ),
        out_shape=jax.ShapeDtypeStruct((N_DEV * m_per, n_per), jnp.float32),
        compiler_params=pltpu.CompilerParams(
            dimension_semantics=("arbitrary", "arbitrary"),
            collective_id=0,
            vmem_limit_bytes=128 * 1024 * 1024,
        ),
    )(perm, x, w_mat)
